# baseline (speedup 1.0000x reference)
import functools

import jax
import jax.numpy as jnp
from jax import lax
from jax.experimental import pallas as pl
from jax.experimental.pallas import tpu as pltpu
from jax.experimental.pallas import tpu_sc as plsc

N = 10000
E = 320000
D_FEAT = 128
HID = 32
DW = 8

NC = 2
NS = 16
NW = NC * NS
CH = 128
NROWS = E // CH
NFULL = NROWS // NW
NEXTRA = NROWS - NFULL * NW
EXTRA_ROW = NFULL * NW

RM = 624
REXT = N - NS * RM
EXT_OFF = NS * RM

_mesh = plsc.VectorSubcoreMesh(
    core_axis_name="c", subcore_axis_name="s", num_cores=NC, num_subcores=NS)
_sc_params = pltpu.CompilerParams(use_tc_tiling_on_sc=False)


@functools.partial(
    pl.kernel,
    out_type=jax.ShapeDtypeStruct((NC, N, DW), jnp.float32),
    mesh=_mesh,
    scratch_types=[
        pltpu.VMEM((NFULL + 1, CH), jnp.int32),
        pltpu.VMEM((CH, DW), jnp.float32),
        pltpu.VMEM_SHARED((N, DW), jnp.float32),
        pltpu.SemaphoreType.DMA,
        pltpu.SemaphoreType.DMA,
    ],
    compiler_params=_sc_params,
)
def _deg_kernel(e3_hbm, zeros_hbm, ones_hbm, degp_hbm, didx_all, ones_v,
                deg_sh, sem0, sem1):
    cid = lax.axis_index("c")
    sid = lax.axis_index("s")
    wid = cid * NS + sid
    row = pl.ds(sid * RM, RM)
    ext = pl.ds(EXT_OFF, REXT)
    pltpu.sync_copy(zeros_hbm.at[row], deg_sh.at[row])

    @pl.when(sid == NS - 1)
    def _():
        pltpu.sync_copy(zeros_hbm.at[ext], deg_sh.at[ext])

    pltpu.sync_copy(ones_hbm, ones_v)
    pltpu.sync_copy(e3_hbm.at[1, pl.ds(wid * NFULL, NFULL)],
                    didx_all.at[pl.ds(0, NFULL)])

    @pl.when(wid < NEXTRA)
    def _():
        pltpu.sync_copy(e3_hbm.at[1, EXTRA_ROW + wid], didx_all.at[NFULL])

    plsc.subcore_barrier()

    def body(j, carry):
        c0 = pltpu.async_copy(ones_v, deg_sh.at[didx_all.at[2 * j]], sem0,
                              add=True)
        c1 = pltpu.async_copy(ones_v, deg_sh.at[didx_all.at[2 * j + 1]], sem1,
                              add=True)
        c0.wait()
        c1.wait()
        return carry

    lax.fori_loop(0, NFULL // 2, body, 0)

    @pl.when(wid < NEXTRA)
    def _():
        pltpu.sync_copy(ones_v, deg_sh.at[didx_all.at[NFULL]], add=True)

    plsc.subcore_barrier()
    pltpu.sync_copy(deg_sh.at[row], degp_hbm.at[cid, row])

    @pl.when(sid == NS - 1)
    def _():
        pltpu.sync_copy(deg_sh.at[ext], degp_hbm.at[cid, ext])


@functools.partial(
    pl.kernel,
    out_type=jax.ShapeDtypeStruct((NC, N, HID), jnp.float32),
    mesh=_mesh,
    scratch_types=[
        pltpu.VMEM((NFULL + 1, CH), jnp.int32),
        pltpu.VMEM((NFULL + 1, CH), jnp.int32),
        pltpu.VMEM((4, CH, HID), jnp.float32),
        pltpu.VMEM_SHARED((N, HID), jnp.float32),
        pltpu.SemaphoreType.DMA,
        pltpu.SemaphoreType.DMA,
        pltpu.SemaphoreType.DMA,
        pltpu.SemaphoreType.DMA,
        pltpu.SemaphoreType.DMA,
        pltpu.SemaphoreType.DMA,
        pltpu.SemaphoreType.DMA,
        pltpu.SemaphoreType.DMA,
    ],
    compiler_params=_sc_params,
)
def _msg_kernel(g_hbm, e3_hbm, out_hbm,
                sidx_all, didx_all, rows, acc_sh,
                gs0, gs1, gs2, gs3, ss0, ss1, ss2, ss3):
    cid = lax.axis_index("c")
    sid = lax.axis_index("s")
    wid = cid * NS + sid
    row = pl.ds(sid * RM, RM)
    ext = pl.ds(EXT_OFF, REXT)
    pltpu.sync_copy(g_hbm.at[row], acc_sh.at[row])

    @pl.when(sid == NS - 1)
    def _():
        pltpu.sync_copy(g_hbm.at[ext], acc_sh.at[ext])

    pltpu.sync_copy(e3_hbm.at[0, pl.ds(wid * NFULL, NFULL)],
                    sidx_all.at[pl.ds(0, NFULL)])
    pltpu.sync_copy(e3_hbm.at[1, pl.ds(wid * NFULL, NFULL)],
                    didx_all.at[pl.ds(0, NFULL)])

    @pl.when(wid < NEXTRA)
    def _():
        pltpu.sync_copy(e3_hbm.at[0, EXTRA_ROW + wid], sidx_all.at[NFULL])
        pltpu.sync_copy(e3_hbm.at[1, EXTRA_ROW + wid], didx_all.at[NFULL])

    plsc.subcore_barrier()
    gsems = (gs0, gs1, gs2, gs3)
    ssems = (ss0, ss1, ss2, ss3)

    def body(q, carry):
        base = 4 * q
        cg = [pltpu.async_copy(g_hbm.at[sidx_all.at[base + b]], rows.at[b],
                               gsems[b]) for b in range(4)]
        cs = []
        for b in range(4):
            cg[b].wait()
            cs.append(pltpu.async_copy(rows.at[b],
                                       acc_sh.at[didx_all.at[base + b]],
                                       ssems[b], add=True))
        for b in range(4):
            cs[b].wait()
        return carry

    NQ = NFULL // 4
    lax.fori_loop(0, NQ, body, 0)

    for r in range(NQ * 4, NFULL):
        b = r - NQ * 4
        pltpu.async_copy(g_hbm.at[sidx_all.at[r]], rows.at[b], gsems[b]).wait()
        pltpu.sync_copy(rows.at[b], acc_sh.at[didx_all.at[r]], add=True)

    @pl.when(wid < NEXTRA)
    def _():
        pltpu.async_copy(g_hbm.at[sidx_all.at[NFULL]], rows.at[0], gs0).wait()
        pltpu.sync_copy(rows.at[0], acc_sh.at[didx_all.at[NFULL]], add=True)

    plsc.subcore_barrier()
    pltpu.sync_copy(acc_sh.at[row], out_hbm.at[cid, row])

    @pl.when(sid == NS - 1)
    def _():
        pltpu.sync_copy(acc_sh.at[ext], out_hbm.at[cid, ext])


R = 2000


def _mm_body(x_ref, w_ref, t_ref):
    t_ref[...] = jnp.dot(x_ref[...], w_ref[...],
                         preferred_element_type=jnp.float32)


def _mm_call(x, W0):
    return pl.pallas_call(
        _mm_body,
        grid=(N // R,),
        in_specs=[
            pl.BlockSpec((R, D_FEAT), lambda i: (i, 0)),
            pl.BlockSpec((D_FEAT, HID), lambda i: (0, 0)),
        ],
        out_specs=pl.BlockSpec((R, HID), lambda i: (i, 0)),
        out_shape=jax.ShapeDtypeStruct((N, HID), jnp.float32),
    )(x, W0)


def _ab_body(t_ref, degp_ref, dinv_ref, g_ref):
    deg = degp_ref[0, :, 0:1] + degp_ref[1, :, 0:1] + 1.0
    dinv = lax.rsqrt(deg)
    dinv_ref[...] = dinv
    g_ref[...] = t_ref[...] * dinv


def _ab_call(t0, degp):
    return pl.pallas_call(
        _ab_body,
        grid=(N // R,),
        in_specs=[
            pl.BlockSpec((R, HID), lambda i: (i, 0)),
            pl.BlockSpec((NC, R, DW), lambda i: (0, i, 0)),
        ],
        out_specs=[
            pl.BlockSpec((R, 1), lambda i: (i, 0)),
            pl.BlockSpec((R, HID), lambda i: (i, 0)),
        ],
        out_shape=[
            jax.ShapeDtypeStruct((N, 1), jnp.float32),
            jax.ShapeDtypeStruct((N, HID), jnp.float32),
        ],
    )(t0, degp)


def _layer_body(p_ref, g_ref, dinv_ref, b_ref, w_ref, gnew_ref):
    dinv = dinv_ref[...]
    s = p_ref[0] + p_ref[1] - g_ref[...]
    h = jnp.maximum(s * dinv + b_ref[...], 0.0)
    t = jnp.dot(h, w_ref[...], preferred_element_type=jnp.float32)
    gnew_ref[...] = t * dinv


def _layer_call(p, g, dinv, b, W):
    return pl.pallas_call(
        _layer_body,
        grid=(N // R,),
        in_specs=[
            pl.BlockSpec((NC, R, HID), lambda i: (0, i, 0)),
            pl.BlockSpec((R, HID), lambda i: (i, 0)),
            pl.BlockSpec((R, 1), lambda i: (i, 0)),
            pl.BlockSpec((1, HID), lambda i: (0, 0)),
            pl.BlockSpec((HID, HID), lambda i: (0, 0)),
        ],
        out_specs=pl.BlockSpec((R, HID), lambda i: (i, 0)),
        out_shape=jax.ShapeDtypeStruct((N, HID), jnp.float32),
    )(p, g, dinv, b, W)


def _head_body(p_ref, g_ref, dinv_ref, b2_ref, wl1_ref, bl1_ref, wl2_ref,
               bl2_ref, out_ref):
    dinv = dinv_ref[...]
    s = p_ref[0] + p_ref[1] - g_ref[...]
    h = jnp.maximum(s * dinv + b2_ref[...], 0.0)
    h = jnp.maximum(
        jnp.dot(h, wl1_ref[...], preferred_element_type=jnp.float32)
        + bl1_ref[...], 0.0)
    out_ref[...] = (
        jnp.dot(h, wl2_ref[...], preferred_element_type=jnp.float32)
        + bl2_ref[...])


def _head_call(p, g, dinv, b2, Wl1, bl1, Wl2, bl2):
    return pl.pallas_call(
        _head_body,
        grid=(N // R,),
        in_specs=[
            pl.BlockSpec((NC, R, HID), lambda i: (0, i, 0)),
            pl.BlockSpec((R, HID), lambda i: (i, 0)),
            pl.BlockSpec((R, 1), lambda i: (i, 0)),
            pl.BlockSpec((1, HID), lambda i: (0, 0)),
            pl.BlockSpec((HID, HID), lambda i: (0, 0)),
            pl.BlockSpec((1, HID), lambda i: (0, 0)),
            pl.BlockSpec((HID, 1), lambda i: (0, 0)),
            pl.BlockSpec((1, 1), lambda i: (0, 0)),
        ],
        out_specs=pl.BlockSpec((R, 1), lambda i: (i, 0)),
        out_shape=jax.ShapeDtypeStruct((N, 1), jnp.float32),
    )(p, g, dinv, b2, Wl1, bl1, Wl2, bl2)


def kernel(x, edge_index, W0, b0, W1, b1, W2, b2, Wl1, bl1, Wl2, bl2):
    e3 = edge_index.reshape(2, NROWS, CH)
    zeros = jnp.zeros((N, DW), jnp.float32)
    ones = jnp.ones((CH, DW), jnp.float32)
    b0r = b0.reshape(1, HID)
    b1r = b1.reshape(1, HID)
    b2r = b2.reshape(1, HID)
    bl1r = bl1.reshape(1, HID)
    bl2r = bl2.reshape(1, 1)
    Wl2r = Wl2.reshape(HID, 1)

    t0 = _mm_call(x, W0)
    degp = _deg_kernel(e3, zeros, ones)
    dinv, g0 = _ab_call(t0, degp)
    p1 = _msg_kernel(g0, e3)
    g1 = _layer_call(p1, g0, dinv, b0r, W1)
    p2 = _msg_kernel(g1, e3)
    g2 = _layer_call(p2, g1, dinv, b1r, W2)
    p3 = _msg_kernel(g2, e3)
    out = _head_call(p3, g2, dinv, b2r, Wl1, bl1r, Wl2r, bl2r)
    return out


# speedup vs baseline: 1.0350x; 1.0350x over previous
import functools

import jax
import jax.numpy as jnp
from jax import lax
from jax.experimental import pallas as pl
from jax.experimental.pallas import tpu as pltpu
from jax.experimental.pallas import tpu_sc as plsc

N = 10000
E = 320000
D_FEAT = 128
HID = 32
DW = 8

NC = 2
NS = 16
NW = NC * NS
CH = 128
NROWS = E // CH
NFULL = NROWS // NW
NEXTRA = NROWS - NFULL * NW
EXTRA_ROW = NFULL * NW

RM = 624
REXT = N - NS * RM
EXT_OFF = NS * RM

_mesh = plsc.VectorSubcoreMesh(
    core_axis_name="c", subcore_axis_name="s", num_cores=NC, num_subcores=NS)
_sc_params = pltpu.CompilerParams(use_tc_tiling_on_sc=False)


@functools.partial(
    pl.kernel,
    out_type=jax.ShapeDtypeStruct((NC, N, DW), jnp.float32),
    mesh=_mesh,
    scratch_types=[
        pltpu.VMEM((NFULL + 1, CH), jnp.int32),
        pltpu.VMEM((CH, DW), jnp.float32),
        pltpu.VMEM_SHARED((N, DW), jnp.float32),
        pltpu.SemaphoreType.DMA,
        pltpu.SemaphoreType.DMA,
    ],
    compiler_params=_sc_params,
)
def _deg_kernel(e3_hbm, zeros_hbm, ones_hbm, degp_hbm, didx_all, ones_v,
                deg_sh, sem0, sem1):
    cid = lax.axis_index("c")
    sid = lax.axis_index("s")
    wid = cid * NS + sid
    row = pl.ds(sid * RM, RM)
    ext = pl.ds(EXT_OFF, REXT)
    pltpu.sync_copy(zeros_hbm.at[row], deg_sh.at[row])

    @pl.when(sid == NS - 1)
    def _():
        pltpu.sync_copy(zeros_hbm.at[ext], deg_sh.at[ext])

    pltpu.sync_copy(ones_hbm, ones_v)
    pltpu.sync_copy(e3_hbm.at[1, pl.ds(wid * NFULL, NFULL)],
                    didx_all.at[pl.ds(0, NFULL)])

    @pl.when(wid < NEXTRA)
    def _():
        pltpu.sync_copy(e3_hbm.at[1, EXTRA_ROW + wid], didx_all.at[NFULL])

    plsc.subcore_barrier()

    def body(j, carry):
        c0 = pltpu.async_copy(ones_v, deg_sh.at[didx_all.at[2 * j]], sem0,
                              add=True)
        c1 = pltpu.async_copy(ones_v, deg_sh.at[didx_all.at[2 * j + 1]], sem1,
                              add=True)
        c0.wait()
        c1.wait()
        return carry

    lax.fori_loop(0, NFULL // 2, body, 0)

    @pl.when(wid < NEXTRA)
    def _():
        pltpu.sync_copy(ones_v, deg_sh.at[didx_all.at[NFULL]], add=True)

    plsc.subcore_barrier()
    pltpu.sync_copy(deg_sh.at[row], degp_hbm.at[cid, row])

    @pl.when(sid == NS - 1)
    def _():
        pltpu.sync_copy(deg_sh.at[ext], degp_hbm.at[cid, ext])


@functools.partial(
    pl.kernel,
    out_type=jax.ShapeDtypeStruct((NC, N, HID), jnp.float32),
    mesh=_mesh,
    scratch_types=[
        pltpu.VMEM((NFULL + 1, CH), jnp.int32),
        pltpu.VMEM((NFULL + 1, CH), jnp.int32),
        pltpu.VMEM((8, CH, HID), jnp.float32),
        pltpu.VMEM_SHARED((N, HID), jnp.float32),
        pltpu.SemaphoreType.DMA,
        pltpu.SemaphoreType.DMA,
        pltpu.SemaphoreType.DMA,
        pltpu.SemaphoreType.DMA,
        pltpu.SemaphoreType.DMA,
        pltpu.SemaphoreType.DMA,
        pltpu.SemaphoreType.DMA,
        pltpu.SemaphoreType.DMA,
        pltpu.SemaphoreType.DMA,
        pltpu.SemaphoreType.DMA,
        pltpu.SemaphoreType.DMA,
        pltpu.SemaphoreType.DMA,
        pltpu.SemaphoreType.DMA,
        pltpu.SemaphoreType.DMA,
        pltpu.SemaphoreType.DMA,
        pltpu.SemaphoreType.DMA,
    ],
    compiler_params=_sc_params,
)
def _msg_kernel(g_hbm, e3_hbm, out_hbm,
                sidx_all, didx_all, rows, acc_sh,
                gs0, gs1, gs2, gs3, gs4, gs5, gs6, gs7,
                ss0, ss1, ss2, ss3, ss4, ss5, ss6, ss7):
    cid = lax.axis_index("c")
    sid = lax.axis_index("s")
    wid = cid * NS + sid
    row = pl.ds(sid * RM, RM)
    ext = pl.ds(EXT_OFF, REXT)
    pltpu.sync_copy(g_hbm.at[row], acc_sh.at[row])

    @pl.when(sid == NS - 1)
    def _():
        pltpu.sync_copy(g_hbm.at[ext], acc_sh.at[ext])

    pltpu.sync_copy(e3_hbm.at[0, pl.ds(wid * NFULL, NFULL)],
                    sidx_all.at[pl.ds(0, NFULL)])
    pltpu.sync_copy(e3_hbm.at[1, pl.ds(wid * NFULL, NFULL)],
                    didx_all.at[pl.ds(0, NFULL)])

    @pl.when(wid < NEXTRA)
    def _():
        pltpu.sync_copy(e3_hbm.at[0, EXTRA_ROW + wid], sidx_all.at[NFULL])
        pltpu.sync_copy(e3_hbm.at[1, EXTRA_ROW + wid], didx_all.at[NFULL])

    plsc.subcore_barrier()
    gsems = (gs0, gs1, gs2, gs3, gs4, gs5, gs6, gs7)
    ssems = (ss0, ss1, ss2, ss3, ss4, ss5, ss6, ss7)

    def body(q, carry):
        base = 8 * q
        cg = [pltpu.async_copy(g_hbm.at[sidx_all.at[base + b]], rows.at[b],
                               gsems[b]) for b in range(8)]
        cs = []
        for b in range(8):
            cg[b].wait()
            cs.append(pltpu.async_copy(rows.at[b],
                                       acc_sh.at[didx_all.at[base + b]],
                                       ssems[b], add=True))
        for b in range(8):
            cs[b].wait()
        return carry

    NQ = NFULL // 8
    lax.fori_loop(0, NQ, body, 0)

    for r in range(NQ * 8, NFULL):
        b = r - NQ * 8
        pltpu.async_copy(g_hbm.at[sidx_all.at[r]], rows.at[b], gsems[b]).wait()
        pltpu.sync_copy(rows.at[b], acc_sh.at[didx_all.at[r]], add=True)

    @pl.when(wid < NEXTRA)
    def _():
        pltpu.async_copy(g_hbm.at[sidx_all.at[NFULL]], rows.at[0], gs0).wait()
        pltpu.sync_copy(rows.at[0], acc_sh.at[didx_all.at[NFULL]], add=True)

    plsc.subcore_barrier()
    pltpu.sync_copy(acc_sh.at[row], out_hbm.at[cid, row])

    @pl.when(sid == NS - 1)
    def _():
        pltpu.sync_copy(acc_sh.at[ext], out_hbm.at[cid, ext])


R = 2000


def _mm_body(x_ref, w_ref, t_ref):
    t_ref[...] = jnp.dot(x_ref[...], w_ref[...],
                         preferred_element_type=jnp.float32)


def _mm_call(x, W0):
    return pl.pallas_call(
        _mm_body,
        grid=(N // R,),
        in_specs=[
            pl.BlockSpec((R, D_FEAT), lambda i: (i, 0)),
            pl.BlockSpec((D_FEAT, HID), lambda i: (0, 0)),
        ],
        out_specs=pl.BlockSpec((R, HID), lambda i: (i, 0)),
        out_shape=jax.ShapeDtypeStruct((N, HID), jnp.float32),
    )(x, W0)


def _ab_body(t_ref, degp_ref, dinv_ref, g_ref):
    deg = degp_ref[0, :, 0:1] + degp_ref[1, :, 0:1] + 1.0
    dinv = lax.rsqrt(deg)
    dinv_ref[...] = dinv
    g_ref[...] = t_ref[...] * dinv


def _ab_call(t0, degp):
    return pl.pallas_call(
        _ab_body,
        grid=(N // R,),
        in_specs=[
            pl.BlockSpec((R, HID), lambda i: (i, 0)),
            pl.BlockSpec((NC, R, DW), lambda i: (0, i, 0)),
        ],
        out_specs=[
            pl.BlockSpec((R, 1), lambda i: (i, 0)),
            pl.BlockSpec((R, HID), lambda i: (i, 0)),
        ],
        out_shape=[
            jax.ShapeDtypeStruct((N, 1), jnp.float32),
            jax.ShapeDtypeStruct((N, HID), jnp.float32),
        ],
    )(t0, degp)


def _layer_body(p_ref, g_ref, dinv_ref, b_ref, w_ref, gnew_ref):
    dinv = dinv_ref[...]
    s = p_ref[0] + p_ref[1] - g_ref[...]
    h = jnp.maximum(s * dinv + b_ref[...], 0.0)
    t = jnp.dot(h, w_ref[...], preferred_element_type=jnp.float32)
    gnew_ref[...] = t * dinv


def _layer_call(p, g, dinv, b, W):
    return pl.pallas_call(
        _layer_body,
        grid=(N // R,),
        in_specs=[
            pl.BlockSpec((NC, R, HID), lambda i: (0, i, 0)),
            pl.BlockSpec((R, HID), lambda i: (i, 0)),
            pl.BlockSpec((R, 1), lambda i: (i, 0)),
            pl.BlockSpec((1, HID), lambda i: (0, 0)),
            pl.BlockSpec((HID, HID), lambda i: (0, 0)),
        ],
        out_specs=pl.BlockSpec((R, HID), lambda i: (i, 0)),
        out_shape=jax.ShapeDtypeStruct((N, HID), jnp.float32),
    )(p, g, dinv, b, W)


def _head_body(p_ref, g_ref, dinv_ref, b2_ref, wl1_ref, bl1_ref, wl2_ref,
               bl2_ref, out_ref):
    dinv = dinv_ref[...]
    s = p_ref[0] + p_ref[1] - g_ref[...]
    h = jnp.maximum(s * dinv + b2_ref[...], 0.0)
    h = jnp.maximum(
        jnp.dot(h, wl1_ref[...], preferred_element_type=jnp.float32)
        + bl1_ref[...], 0.0)
    out_ref[...] = (
        jnp.dot(h, wl2_ref[...], preferred_element_type=jnp.float32)
        + bl2_ref[...])


def _head_call(p, g, dinv, b2, Wl1, bl1, Wl2, bl2):
    return pl.pallas_call(
        _head_body,
        grid=(N // R,),
        in_specs=[
            pl.BlockSpec((NC, R, HID), lambda i: (0, i, 0)),
            pl.BlockSpec((R, HID), lambda i: (i, 0)),
            pl.BlockSpec((R, 1), lambda i: (i, 0)),
            pl.BlockSpec((1, HID), lambda i: (0, 0)),
            pl.BlockSpec((HID, HID), lambda i: (0, 0)),
            pl.BlockSpec((1, HID), lambda i: (0, 0)),
            pl.BlockSpec((HID, 1), lambda i: (0, 0)),
            pl.BlockSpec((1, 1), lambda i: (0, 0)),
        ],
        out_specs=pl.BlockSpec((R, 1), lambda i: (i, 0)),
        out_shape=jax.ShapeDtypeStruct((N, 1), jnp.float32),
    )(p, g, dinv, b2, Wl1, bl1, Wl2, bl2)


def kernel(x, edge_index, W0, b0, W1, b1, W2, b2, Wl1, bl1, Wl2, bl2):
    e3 = edge_index.reshape(2, NROWS, CH)
    zeros = jnp.zeros((N, DW), jnp.float32)
    ones = jnp.ones((CH, DW), jnp.float32)
    b0r = b0.reshape(1, HID)
    b1r = b1.reshape(1, HID)
    b2r = b2.reshape(1, HID)
    bl1r = bl1.reshape(1, HID)
    bl2r = bl2.reshape(1, 1)
    Wl2r = Wl2.reshape(HID, 1)

    t0 = _mm_call(x, W0)
    degp = _deg_kernel(e3, zeros, ones)
    dinv, g0 = _ab_call(t0, degp)
    p1 = _msg_kernel(g0, e3)
    g1 = _layer_call(p1, g0, dinv, b0r, W1)
    p2 = _msg_kernel(g1, e3)
    g2 = _layer_call(p2, g1, dinv, b1r, W2)
    p3 = _msg_kernel(g2, e3)
    out = _head_call(p3, g2, dinv, b2r, Wl1, bl1r, Wl2r, bl2r)
    return out
